# Initial kernel scaffold; baseline (speedup 1.0000x reference)
import jax
import jax.numpy as jnp
from jax.experimental import pallas as pl


def kernel(x, reference_inputs, reference_outputs):
    raise NotImplementedError("write your pallas kernel here")


# speedup vs baseline: 10.9063x; 10.9063x over previous
import functools

import jax
import jax.numpy as jnp
from jax import lax
from jax.experimental import pallas as pl
from jax.experimental.pallas import tpu as pltpu, tpu_sc as plsc

R = 10000
B = 16384
NC, NS, L = 2, 16, 16
NW = NC * NS
BPW = B // NW
STEPS = 14


def _calib_body(x_hbm, ri_hbm, ro_hbm, out_hbm, ri_v, ro_v, x_v, o_v, sem):
    wid = lax.axis_index("s") * NC + lax.axis_index("c")
    base = wid * BPW

    pltpu.sync_copy(ri_hbm, ri_v)
    pltpu.sync_copy(ro_hbm, ro_v)
    pltpu.sync_copy(x_hbm.at[pl.ds(base, BPW)], x_v)

    zeros = jnp.zeros((L,), jnp.int32)
    last = jnp.full((L,), R - 1, jnp.int32)
    ri_first = plsc.load_gather(ri_v, [zeros])
    ri_last = plsc.load_gather(ri_v, [last])
    ro_first = plsc.load_gather(ro_v, [zeros])
    ro_last = plsc.load_gather(ro_v, [last])

    def body(i, carry):
        xx = x_v[pl.ds(i * L, L)]
        lo = jnp.zeros((L,), jnp.int32)
        hi = jnp.full((L,), R, jnp.int32)
        for _ in range(STEPS):
            mid = lax.shift_right_arithmetic(lo + hi, 1)
            v = plsc.load_gather(ri_v, [jnp.minimum(mid, R - 1)])
            gt = v > xx
            lo = jnp.where(gt, lo, mid + 1)
            hi = jnp.where(gt, mid, hi)
        idx = jnp.minimum(jnp.maximum(lo, 1), R - 1)
        ri_hi = plsc.load_gather(ri_v, [idx])
        ri_lo = plsc.load_gather(ri_v, [idx - 1])
        ro_hi = plsc.load_gather(ro_v, [idx])
        ro_lo = plsc.load_gather(ro_v, [idx - 1])
        m = (ro_hi - ro_lo) / (ri_hi - ri_lo)
        interp = ro_lo + m * (xx - ri_lo)
        out = jnp.where(xx >= ri_last, ro_last,
                        jnp.where(xx <= ri_first, ro_first, interp))
        o_v[pl.ds(i * L, L)] = out
        return carry

    lax.fori_loop(0, BPW // L, body, 0)
    pltpu.sync_copy(o_v, out_hbm.at[pl.ds(base, BPW)])


def kernel(x, reference_inputs, reference_outputs):
    mesh = plsc.VectorSubcoreMesh(core_axis_name="c", subcore_axis_name="s")
    run = functools.partial(
        pl.kernel,
        mesh=mesh,
        out_type=jax.ShapeDtypeStruct((B,), jnp.float32),
        scratch_types=[
            pltpu.VMEM((R,), jnp.float32),
            pltpu.VMEM((R,), jnp.float32),
            pltpu.VMEM((BPW,), jnp.float32),
            pltpu.VMEM((BPW,), jnp.float32),
            pltpu.SemaphoreType.DMA,
        ],
        compiler_params=pltpu.CompilerParams(needs_layout_passes=False),
    )(_calib_body)
    out = run(x[:, 0], reference_inputs, reference_outputs)
    return out[:, None]


# speedup vs baseline: 12.4193x; 1.1387x over previous
import functools

import jax
import jax.numpy as jnp
from jax import lax
from jax.experimental import pallas as pl
from jax.experimental.pallas import tpu as pltpu, tpu_sc as plsc

R = 10000
B = 16384
NC, NS, L = 2, 16, 16
NW = NC * NS
BPW = B // NW
STEPS = 14


def _calib_body(x_hbm, ri_hbm, ro_hbm, out_hbm, ri_v, ro_v, x_v, o_v, sem):
    wid = lax.axis_index("s") * NC + lax.axis_index("c")
    base = wid * BPW

    c1 = pltpu.async_copy(ri_hbm, ri_v, sem)
    c2 = pltpu.async_copy(ro_hbm, ro_v, sem)
    c3 = pltpu.async_copy(x_hbm.at[pl.ds(base, BPW)], x_v, sem)
    c1.wait()
    c2.wait()
    c3.wait()

    zeros = jnp.zeros((L,), jnp.int32)
    last = jnp.full((L,), R - 1, jnp.int32)
    ri_first = plsc.load_gather(ri_v, [zeros])
    ri_last = plsc.load_gather(ri_v, [last])
    ro_first = plsc.load_gather(ro_v, [zeros])
    ro_last = plsc.load_gather(ro_v, [last])

    @plsc.parallel_loop(0, BPW // L, unroll=4)
    def body(i):
        xx = x_v[pl.ds(i * L, L)]
        lo = jnp.zeros((L,), jnp.int32)
        hi = jnp.full((L,), R, jnp.int32)
        for _ in range(STEPS):
            mid = lax.shift_right_arithmetic(lo + hi, 1)
            v = plsc.load_gather(ri_v, [jnp.minimum(mid, R - 1)])
            gt = v > xx
            lo = jnp.where(gt, lo, mid + 1)
            hi = jnp.where(gt, mid, hi)
        idx = jnp.minimum(jnp.maximum(lo, 1), R - 1)
        ri_hi = plsc.load_gather(ri_v, [idx])
        ri_lo = plsc.load_gather(ri_v, [idx - 1])
        ro_hi = plsc.load_gather(ro_v, [idx])
        ro_lo = plsc.load_gather(ro_v, [idx - 1])
        m = (ro_hi - ro_lo) / (ri_hi - ri_lo)
        interp = ro_lo + m * (xx - ri_lo)
        out = jnp.where(xx >= ri_last, ro_last,
                        jnp.where(xx <= ri_first, ro_first, interp))
        o_v[pl.ds(i * L, L)] = out

    pltpu.sync_copy(o_v, out_hbm.at[pl.ds(base, BPW)])


def kernel(x, reference_inputs, reference_outputs):
    mesh = plsc.VectorSubcoreMesh(core_axis_name="c", subcore_axis_name="s")
    run = functools.partial(
        pl.kernel,
        mesh=mesh,
        out_type=jax.ShapeDtypeStruct((B,), jnp.float32),
        scratch_types=[
            pltpu.VMEM((R,), jnp.float32),
            pltpu.VMEM((R,), jnp.float32),
            pltpu.VMEM((BPW,), jnp.float32),
            pltpu.VMEM((BPW,), jnp.float32),
            pltpu.SemaphoreType.DMA,
        ],
        compiler_params=pltpu.CompilerParams(needs_layout_passes=False),
    )(_calib_body)
    out = run(x[:, 0], reference_inputs, reference_outputs)
    return out[:, None]
